# baseline (speedup 1.0000x reference)
import functools

import jax
import jax.numpy as jnp
from jax import lax
from jax.experimental import pallas as pl
from jax.experimental.pallas import tpu as pltpu
from jax.experimental.pallas import tpu_sc as plsc

L = 16
S = 32
D = 128
NBUF = 4
R_C0 = 216
R_C1 = 424
R_MAX = max(R_C0, R_C1)
PAD_B = 16 * (R_C0 + R_C1)


_DG_DNUMS = lax.GatherDimensionNumbers(
    offset_dims=(), collapsed_slice_dims=(0,), start_index_map=(0,))


def _dg(x, idx):
    return lax.gather(x, idx[:, None], _DG_DNUMS, slice_sizes=(1,),
                      mode=lax.GatherScatterMode.PROMISE_IN_BOUNDS)


def _row_weights(idx_ref, base):
    a = idx_ref[pl.ds(base, L)]
    b = idx_ref[pl.ds(base + L, L)]
    iota = lax.iota(jnp.int32, L)
    ca = jnp.ones((L,), jnp.float32)
    cb = jnp.ones((L,), jnp.float32)
    for r in range(L):
        ir = (iota + r) & (L - 1)
        ra = _dg(a, ir) if r else a
        rb = _dg(b, ir) if r else b
        if r:
            ca = jnp.where(a == ra, ca + 1.0, ca)
            cb = jnp.where(b == rb, cb + 1.0, cb)
        ca = jnp.where(a == rb, ca + 1.0, ca)
        cb = jnp.where(b == ra, cb + 1.0, cb)
    wa = 1.0 / ca
    wb = 1.0 / cb
    s = wa + wb
    for sh in (1, 2, 4, 8):
        s = s + _dg(s, (iota + sh) & (L - 1))
    inv_n = 1.0 / s
    return wa * inv_n, wb * inv_n


_mesh = plsc.VectorSubcoreMesh(core_axis_name="c", subcore_axis_name="s")


@functools.partial(
    pl.kernel,
    out_type=jax.ShapeDtypeStruct((PAD_B, D), jnp.float32),
    mesh=_mesh,
    scratch_types=[
        pltpu.VMEM((R_MAX * S,), jnp.int32),
        pltpu.VMEM((S, D), jnp.float32),
        pltpu.VMEM((S, D), jnp.float32),
        pltpu.VMEM((S, D), jnp.float32),
        pltpu.VMEM((S, D), jnp.float32),
        pltpu.VMEM((R_MAX, D), jnp.float32),
        pltpu.SemaphoreType.DMA,
        pltpu.SemaphoreType.DMA,
        pltpu.SemaphoreType.DMA,
        pltpu.SemaphoreType.DMA,
    ],
)
def _sc_agg(table_hbm, idx_hbm, out_hbm, idx_v, buf0, buf1, buf2, buf3,
            out_v, sem0, sem1, sem2, sem3):
    cid = lax.axis_index("c")
    sid = lax.axis_index("s")
    bufs = ((buf0, sem0), (buf1, sem1), (buf2, sem2), (buf3, sem3))

    def gather_row(i, buf, sem):
        off = pl.multiple_of(i * S, S)
        pltpu.async_copy(table_hbm.at[idx_v.at[pl.ds(off, S)]], buf, sem)

    def wait_row(i, buf, sem):
        off = pl.multiple_of(i * S, S)
        pltpu.make_async_copy(
            table_hbm.at[idx_v.at[pl.ds(off, S)]], buf, sem).wait()

    def run_core(n_rows, base):
        pltpu.sync_copy(idx_hbm.at[pl.ds(base * S, n_rows * S)],
                        idx_v.at[pl.ds(0, n_rows * S)])
        for b, (buf, sem) in enumerate(bufs):
            gather_row(b, buf, sem)

        def step(g, carry):
            for b, (buf, sem) in enumerate(bufs):
                i = NBUF * g + b
                wait_row(i, buf, sem)
                wa, wb = _row_weights(idx_v, pl.multiple_of(i * S, S))
                accs = [jnp.zeros((L,), jnp.float32) for _ in range(D // L)]
                for s in range(S):
                    wv = wa if s < L else wb
                    ws = _dg(wv, jnp.full((L,), s % L, jnp.int32))
                    for c in range(D // L):
                        accs[c] = accs[c] + buf[s, pl.ds(c * L, L)] * ws

                @pl.when(i + NBUF < n_rows)
                def _():
                    gather_row(i + NBUF, buf, sem)

                for c in range(D // L):
                    out_v[i, pl.ds(c * L, L)] = accs[c]
            return carry

        lax.fori_loop(0, n_rows // NBUF, step, 0)
        pltpu.sync_copy(out_v.at[pl.ds(0, n_rows)],
                        out_hbm.at[pl.ds(base, n_rows)])

    @pl.when(cid == 0)
    def _():
        run_core(R_C0, pl.multiple_of(sid * R_C0, 8))

    @pl.when(cid == 1)
    def _():
        run_core(R_C1, pl.multiple_of(16 * R_C0 + sid * R_C1, 8))


def kernel(nodes_real, nodes, samp_neighs, feature_table):
    idx = samp_neighs.astype(jnp.int32)
    b = idx.shape[0]
    idx_flat = jnp.pad(idx, ((0, PAD_B - b), (0, 0))).reshape(-1)
    out = _sc_agg(feature_table, idx_flat)
    return out[:b]
